# speedup vs baseline: 1.4760x; 1.4760x over previous
import functools

import jax
import jax.numpy as jnp
from jax import lax
from jax.experimental import pallas as pl
from jax.experimental.pallas import tpu as pltpu
from jax.experimental.pallas import tpu_sc as plsc

DIS = 26
CONT = 13
FEAT = DIS + CONT
VOCAB = 100000
BATCH = 16384
LANES = 16
NW = 32
RPW = BATCH // NW
NCH = RPW // 128
NSL = RPW // LANES


def _sc_body(xt_hbm, tab_hbm, out_hbm, xbuf, idxbuf, gbuf, obuf, sem):
    nc = plsc.get_sparse_core_info().num_cores
    wid = lax.axis_index("s") * nc + lax.axis_index("c")
    base = wid * RPW

    def x_start(i, _):
        pltpu.async_copy(xt_hbm.at[i, pl.ds(base, RPW)], xbuf.at[i], sem)
        return 0

    lax.fori_loop(0, FEAT, x_start, 0)

    def x_wait(i, _):
        pltpu.make_async_copy(
            xt_hbm.at[i, pl.ds(base, RPW)], xbuf.at[i], sem).wait()
        return 0

    lax.fori_loop(0, FEAT, x_wait, 0)

    def field(i, _):
        def idx_slice(s, _):
            v = xbuf[i, pl.ds(s * LANES, LANES)]
            iv = v.astype(jnp.int32) + i * VOCAB
            idxbuf[i, s // 8, pl.ds((s % 8) * LANES, LANES)] = iv
            return 0

        lax.fori_loop(0, NSL, idx_slice, 0)

        def g_start(j, _):
            pltpu.async_copy(tab_hbm.at[idxbuf.at[i, j]], gbuf.at[i, j], sem)
            return 0

        lax.fori_loop(0, NCH, g_start, 0)
        return 0

    lax.fori_loop(0, DIS, field, 0)

    def cont_slice(s, _):
        def addc(k, acc):
            return acc + xbuf[DIS + k, pl.ds(s * LANES, LANES)]

        acc = lax.fori_loop(0, CONT, addc, jnp.zeros((LANES,), jnp.float32))
        obuf[pl.ds(s * LANES, LANES)] = acc
        return 0

    lax.fori_loop(0, NSL, cont_slice, 0)

    def g_wait(i, _):
        def gw(j, _):
            pltpu.make_async_copy(
                tab_hbm.at[idxbuf.at[i, j]], gbuf.at[i, j], sem).wait()
            return 0

        lax.fori_loop(0, NCH, gw, 0)
        return 0

    lax.fori_loop(0, DIS, g_wait, 0)

    def red_slice(s, _):
        j = s // 8
        c = (s % 8) * LANES

        def addf(i, acc):
            return acc + gbuf[i, j, pl.ds(c, LANES)]

        acc = lax.fori_loop(0, DIS, addf, obuf[pl.ds(s * LANES, LANES)])
        sig = 1.0 / (1.0 + jnp.exp(-acc))
        obuf[pl.ds(s * LANES, LANES)] = sig
        return 0

    lax.fori_loop(0, NSL, red_slice, 0)

    pltpu.sync_copy(obuf, out_hbm.at[pl.ds(base, RPW)])


def kernel(X, tables):
    xt = X.T
    tab = tables.reshape(DIS * VOCAB)
    mesh = plsc.VectorSubcoreMesh(core_axis_name="c", subcore_axis_name="s")
    run = functools.partial(
        pl.kernel,
        mesh=mesh,
        out_type=jax.ShapeDtypeStruct((BATCH,), jnp.float32),
        scratch_types=[
            pltpu.VMEM((FEAT, RPW), jnp.float32),
            pltpu.VMEM((DIS, NCH, 128), jnp.int32),
            pltpu.VMEM((DIS, NCH, 128), jnp.float32),
            pltpu.VMEM((RPW,), jnp.float32),
            pltpu.SemaphoreType.DMA,
        ],
    )(_sc_body)
    out = run(xt, tab)
    return out.reshape(BATCH, 1)
